# Initial kernel scaffold; baseline (speedup 1.0000x reference)
import jax
import jax.numpy as jnp
from jax.experimental import pallas as pl


def kernel(x, W_base, b_base, W_gate, lora_A, lora_B):
    raise NotImplementedError("write your pallas kernel here")


# speedup vs baseline: 1.4636x; 1.4636x over previous
import functools

import jax
import jax.numpy as jnp
from jax.experimental import pallas as pl
from jax.experimental.pallas import tpu as pltpu

E = 8
R = 16
ER = E * R
TOP_K = 2
SCALING = 32.0 / 16.0


def _fused_body(x_ref, wb_ref, b_ref, wg_ref, af_ref, bt_ref, out_ref, hw_ref):
    j = pl.program_id(1)
    tm = x_ref.shape[0]

    @pl.when(j == 0)
    def _routing():
        xt = x_ref[...]
        logits = jax.lax.dot_general(
            xt, wg_ref[...], (((1,), (1,)), ((), ())),
            preferred_element_type=jnp.float32)
        m = jnp.max(logits, axis=1, keepdims=True)
        p = jnp.exp(logits - m)
        p = p / jnp.sum(p, axis=1, keepdims=True)
        eidx = jax.lax.broadcasted_iota(jnp.int32, (tm, E), 1)
        m1 = jnp.max(p, axis=1, keepdims=True)
        i1 = jnp.min(jnp.where(p == m1, eidx, E), axis=1, keepdims=True)
        p2 = jnp.where(eidx == i1, -jnp.inf, p)
        m2 = jnp.max(p2, axis=1, keepdims=True)
        i2 = jnp.min(jnp.where(p2 == m2, eidx, E), axis=1, keepdims=True)
        s = m1 + m2
        w1 = (m1 / s) * SCALING
        w2 = (m2 / s) * SCALING
        h = jax.lax.dot_general(
            xt, af_ref[...], (((1,), (1,)), ((), ())),
            preferred_element_type=jnp.float32)
        cidx = jax.lax.broadcasted_iota(jnp.int32, (tm, ER), 1)
        ec = cidx // R
        cwe = jnp.where(ec == i1, w1, 0.0) + jnp.where(ec == i2, w2, 0.0)
        hw_ref[...] = h * cwe

    acc = jax.lax.dot_general(
        x_ref[...], wb_ref[...], (((1,), (1,)), ((), ())),
        preferred_element_type=jnp.float32)
    acc += jax.lax.dot_general(
        hw_ref[...], bt_ref[...], (((1,), (1,)), ((), ())),
        preferred_element_type=jnp.float32)
    out_ref[...] = acc + b_ref[...]


@functools.partial(jax.jit, static_argnames=("tm", "tn"))
def _run(xf, W_base, b2, W_gate, A_flat, Bt, tm, tn):
    T, D = xf.shape
    grid = (T // tm, D // tn)
    return pl.pallas_call(
        _fused_body,
        grid=grid,
        in_specs=[
            pl.BlockSpec((tm, D), lambda i, j: (i, 0)),
            pl.BlockSpec((tn, D), lambda i, j: (j, 0)),
            pl.BlockSpec((1, tn), lambda i, j: (0, j)),
            pl.BlockSpec((E, D), lambda i, j: (0, 0)),
            pl.BlockSpec((ER, D), lambda i, j: (0, 0)),
            pl.BlockSpec((tn, ER), lambda i, j: (j, 0)),
        ],
        out_specs=pl.BlockSpec((tm, tn), lambda i, j: (i, j)),
        out_shape=jax.ShapeDtypeStruct((T, D), jnp.float32),
        scratch_shapes=[pltpu.VMEM((tm, ER), jnp.float32)],
    )(xf, W_base, b2, W_gate, A_flat, Bt)


def kernel(x, W_base, b_base, W_gate, lora_A, lora_B):
    b, s, d = x.shape
    xf = x.reshape(-1, d)
    A_flat = lora_A.reshape(ER, d)
    Bt = lora_B.transpose(1, 0, 2).reshape(d, ER)
    b2 = b_base.reshape(1, d)
    out = _run(xf, W_base, b2, W_gate, A_flat, Bt, tm=512, tn=1024)
    return out.reshape(b, s, d)


# speedup vs baseline: 2.1296x; 1.4550x over previous
import functools

import jax
import jax.numpy as jnp
from jax.experimental import pallas as pl
from jax.experimental.pallas import tpu as pltpu

E = 8
R = 16
ER = E * R
TOP_K = 2
SCALING = 32.0 / 16.0


def _fused_body(x_ref, wb_ref, b_ref, wg_ref, af_ref, bt_ref, out_ref, hw_ref):
    j = pl.program_id(1)
    tm = x_ref.shape[0]

    @pl.when(j == 0)
    def _routing():
        xt = x_ref[...]
        logits = jax.lax.dot_general(
            xt, wg_ref[...], (((1,), (1,)), ((), ())),
            preferred_element_type=jnp.float32)
        m = jnp.max(logits, axis=1, keepdims=True)
        p = jnp.exp(logits - m)
        p = p / jnp.sum(p, axis=1, keepdims=True)
        eidx = jax.lax.broadcasted_iota(jnp.int32, (tm, E), 1)
        m1 = jnp.max(p, axis=1, keepdims=True)
        i1 = jnp.min(jnp.where(p == m1, eidx, E), axis=1, keepdims=True)
        p2 = jnp.where(eidx == i1, -jnp.inf, p)
        m2 = jnp.max(p2, axis=1, keepdims=True)
        i2 = jnp.min(jnp.where(p2 == m2, eidx, E), axis=1, keepdims=True)
        s = m1 + m2
        w1 = (m1 / s) * SCALING
        w2 = (m2 / s) * SCALING
        h = jax.lax.dot_general(
            xt, af_ref[...], (((1,), (1,)), ((), ())),
            preferred_element_type=jnp.float32)
        cidx = jax.lax.broadcasted_iota(jnp.int32, (tm, ER), 1)
        ec = cidx // R
        cwe = jnp.where(ec == i1, w1, 0.0) + jnp.where(ec == i2, w2, 0.0)
        hw_ref[...] = h * cwe

    acc = jax.lax.dot_general(
        x_ref[...], wb_ref[...], (((1,), (1,)), ((), ())),
        preferred_element_type=jnp.float32)
    acc += jax.lax.dot_general(
        hw_ref[...], bt_ref[...], (((1,), (1,)), ((), ())),
        preferred_element_type=jnp.float32)
    out_ref[...] = acc + b_ref[...]


@functools.partial(jax.jit, static_argnames=("tm", "tn"))
def _run(xf, W_base, b2, W_gate, A_flat, Bt, tm, tn):
    T, D = xf.shape
    grid = (T // tm, D // tn)
    return pl.pallas_call(
        _fused_body,
        grid=grid,
        in_specs=[
            pl.BlockSpec((tm, D), lambda i, j: (i, 0)),
            pl.BlockSpec((tn, D), lambda i, j: (j, 0)),
            pl.BlockSpec((1, tn), lambda i, j: (0, j)),
            pl.BlockSpec((E, D), lambda i, j: (0, 0)),
            pl.BlockSpec((ER, D), lambda i, j: (0, 0)),
            pl.BlockSpec((tn, ER), lambda i, j: (j, 0)),
        ],
        out_specs=pl.BlockSpec((tm, tn), lambda i, j: (i, j)),
        out_shape=jax.ShapeDtypeStruct((T, D), jnp.float32),
        scratch_shapes=[pltpu.VMEM((tm, ER), jnp.float32)],
    )(xf, W_base, b2, W_gate, A_flat, Bt)


def kernel(x, W_base, b_base, W_gate, lora_A, lora_B):
    b, s, d = x.shape
    xf = x.reshape(-1, d)
    A_flat = lora_A.reshape(ER, d)
    Bt = lora_B.transpose(1, 0, 2).reshape(d, ER)
    b2 = b_base.reshape(1, d)
    out = _run(xf, W_base, b2, W_gate, A_flat, Bt, tm=512, tn=2048)
    return out.reshape(b, s, d)


# speedup vs baseline: 2.2149x; 1.0400x over previous
import functools

import jax
import jax.numpy as jnp
from jax.experimental import pallas as pl
from jax.experimental.pallas import tpu as pltpu

E = 8
R = 16
ER = E * R
TOP_K = 2
SCALING = 32.0 / 16.0


def _fused_body(x_ref, wb_ref, b_ref, wg_ref, af_ref, bt_ref, out_ref, hw_ref):
    j = pl.program_id(1)
    tm = x_ref.shape[0]

    @pl.when(j == 0)
    def _routing():
        xt = x_ref[...]
        logits = jax.lax.dot_general(
            xt, wg_ref[...], (((1,), (1,)), ((), ())),
            preferred_element_type=jnp.float32)
        m = jnp.max(logits, axis=1, keepdims=True)
        p = jnp.exp(logits - m)
        p = p / jnp.sum(p, axis=1, keepdims=True)
        eidx = jax.lax.broadcasted_iota(jnp.int32, (tm, E), 1)
        m1 = jnp.max(p, axis=1, keepdims=True)
        i1 = jnp.min(jnp.where(p == m1, eidx, E), axis=1, keepdims=True)
        p2 = jnp.where(eidx == i1, -jnp.inf, p)
        m2 = jnp.max(p2, axis=1, keepdims=True)
        i2 = jnp.min(jnp.where(p2 == m2, eidx, E), axis=1, keepdims=True)
        s = m1 + m2
        w1 = (m1 / s) * SCALING
        w2 = (m2 / s) * SCALING
        h = jax.lax.dot_general(
            xt, af_ref[...], (((1,), (1,)), ((), ())),
            preferred_element_type=jnp.float32)
        cidx = jax.lax.broadcasted_iota(jnp.int32, (tm, ER), 1)
        ec = cidx // R
        cwe = jnp.where(ec == i1, w1, 0.0) + jnp.where(ec == i2, w2, 0.0)
        hw_ref[...] = h * cwe

    acc = jax.lax.dot_general(
        x_ref[...], wb_ref[...], (((1,), (1,)), ((), ())),
        preferred_element_type=jnp.float32)
    acc += jax.lax.dot_general(
        hw_ref[...], bt_ref[...], (((1,), (1,)), ((), ())),
        preferred_element_type=jnp.float32)
    out_ref[...] = acc + b_ref[...]


@functools.partial(jax.jit, static_argnames=("tm", "tn"))
def _run(xf, W_base, b2, W_gate, A_flat, Bt, tm, tn):
    T, D = xf.shape
    grid = (T // tm, D // tn)
    return pl.pallas_call(
        _fused_body,
        grid=grid,
        in_specs=[
            pl.BlockSpec((tm, D), lambda i, j: (i, 0)),
            pl.BlockSpec((tn, D), lambda i, j: (j, 0)),
            pl.BlockSpec((1, tn), lambda i, j: (0, j)),
            pl.BlockSpec((E, D), lambda i, j: (0, 0)),
            pl.BlockSpec((ER, D), lambda i, j: (0, 0)),
            pl.BlockSpec((tn, ER), lambda i, j: (j, 0)),
        ],
        out_specs=pl.BlockSpec((tm, tn), lambda i, j: (i, j)),
        out_shape=jax.ShapeDtypeStruct((T, D), jnp.float32),
        scratch_shapes=[pltpu.VMEM((tm, ER), jnp.float32)],
    )(xf, W_base, b2, W_gate, A_flat, Bt)


def kernel(x, W_base, b_base, W_gate, lora_A, lora_B):
    b, s, d = x.shape
    xf = x.reshape(-1, d)
    A_flat = lora_A.reshape(ER, d)
    Bt = lora_B.transpose(1, 0, 2).reshape(d, ER)
    b2 = b_base.reshape(1, d)
    out = _run(xf, W_base, b2, W_gate, A_flat, Bt, tm=1024, tn=2048)
    return out.reshape(b, s, d)
